# Initial kernel scaffold; baseline (speedup 1.0000x reference)
import jax
import jax.numpy as jnp
from jax.experimental import pallas as pl


def kernel(user_age, user_level, item_id, item_cate, item_id_seq, item_cate_seq, userid, neighbor_ids, W_user_age, W_user_level, W_item_id, W_item_cate, W_mem0, W_mem1):
    raise NotImplementedError("write your pallas kernel here")


# speedup vs baseline: 1.6531x; 1.6531x over previous
import functools

import jax
import jax.numpy as jnp
from jax import lax
from jax.experimental import pallas as pl
from jax.experimental.pallas import tpu as pltpu
from jax.experimental.pallas import tpu_sc as plsc

B = 4096
L = 200
NN = 20
KV = 32
MEM = 32

NC = 2
NS = 16
NW = NC * NS
BC = B // NW


def _make_sc_kernel():
    mesh = plsc.VectorSubcoreMesh(core_axis_name="c", subcore_axis_name="s")

    out_type = [
        jax.ShapeDtypeStruct((B, KV), jnp.float32),
        jax.ShapeDtypeStruct((B, KV), jnp.float32),
        jax.ShapeDtypeStruct((B, KV), jnp.float32),
        jax.ShapeDtypeStruct((B, KV), jnp.float32),
        jax.ShapeDtypeStruct((L, B, KV), jnp.float32),
        jax.ShapeDtypeStruct((L, B, KV), jnp.float32),
        jax.ShapeDtypeStruct((B, MEM), jnp.float32),
        jax.ShapeDtypeStruct((B, MEM), jnp.float32),
        jax.ShapeDtypeStruct((NN, B, MEM), jnp.float32),
        jax.ShapeDtypeStruct((NN, B, MEM), jnp.float32),
    ]

    @functools.partial(
        pl.kernel,
        mesh=mesh,
        out_type=out_type,
        compiler_params=pltpu.CompilerParams(use_tc_tiling_on_sc=False),
        scratch_types=[
            pltpu.VMEM((8, BC), jnp.int32),
            pltpu.VMEM((NN, BC), jnp.int32),
            pltpu.VMEM((L, BC), jnp.int32),
            pltpu.VMEM((L, BC), jnp.int32),
            pltpu.VMEM((BC, KV), jnp.float32),
            pltpu.VMEM((BC, KV), jnp.float32),
            pltpu.SemaphoreType.DMA,
            pltpu.SemaphoreType.DMA,
        ],
    )
    def sc_kernel(ua, ul, ii, ic, sid, sct, uid, nb,
                  t_age, t_lvl, t_item, t_cate, t_m0, t_m1,
                  o_ua, o_ub, o_ia, o_ib, o_sa, o_sb, o_u0, o_u1, o_n0, o_n1,
                  ism, inb, isid, isct, r0, r1, sg, sw):
        wid = lax.axis_index("s") * NC + lax.axis_index("c")
        b0 = wid * BC

        pltpu.sync_copy(ua.at[pl.ds(b0, BC)], ism.at[0])
        pltpu.sync_copy(ul.at[pl.ds(b0, BC)], ism.at[1])
        pltpu.sync_copy(ii.at[pl.ds(b0, BC)], ism.at[2])
        pltpu.sync_copy(ic.at[pl.ds(b0, BC)], ism.at[3])
        pltpu.sync_copy(uid.at[pl.ds(b0, BC)], ism.at[4])
        pltpu.sync_copy(nb.at[:, pl.ds(b0, BC)], inb)
        pltpu.sync_copy(sid.at[:, pl.ds(b0, BC)], isid)
        pltpu.sync_copy(sct.at[:, pl.ds(b0, BC)], isct)

        def pair(tab0, idx0, tab1, idx1, dst0, dst1):
            g0 = pltpu.async_copy(tab0.at[idx0], r0, sg)
            g1 = pltpu.async_copy(tab1.at[idx1], r1, sg)
            g0.wait()
            g1.wait()
            w0 = pltpu.async_copy(r0, dst0, sw)
            w1 = pltpu.async_copy(r1, dst1, sw)
            w0.wait()
            w1.wait()

        pair(t_age, ism.at[0], t_lvl, ism.at[1],
             o_ua.at[pl.ds(b0, BC)], o_ub.at[pl.ds(b0, BC)])
        pair(t_item, ism.at[2], t_cate, ism.at[3],
             o_ia.at[pl.ds(b0, BC)], o_ib.at[pl.ds(b0, BC)])
        pair(t_m0, ism.at[4], t_m1, ism.at[4],
             o_u0.at[pl.ds(b0, BC)], o_u1.at[pl.ds(b0, BC)])

        def nbody(l, carry):
            pair(t_m0, inb.at[l], t_m1, inb.at[l],
                 o_n0.at[l].at[pl.ds(b0, BC)],
                 o_n1.at[l].at[pl.ds(b0, BC)])
            return carry
        lax.fori_loop(0, NN, nbody, 0)

        def sbody(l, carry):
            pair(t_item, isid.at[l], t_cate, isct.at[l],
                 o_sa.at[l].at[pl.ds(b0, BC)],
                 o_sb.at[l].at[pl.ds(b0, BC)])
            return carry
        lax.fori_loop(0, L, sbody, 0)

    return sc_kernel


_SC_KERNEL = _make_sc_kernel()


def kernel(user_age, user_level, item_id, item_cate, item_id_seq,
           item_cate_seq, userid, neighbor_ids, W_user_age, W_user_level,
           W_item_id, W_item_cate, W_mem0, W_mem1):
    o_ua, o_ub, o_ia, o_ib, o_sa, o_sb, o_u0, o_u1, o_n0, o_n1 = _SC_KERNEL(
        user_age, user_level, item_id, item_cate,
        item_id_seq.T, item_cate_seq.T, userid, neighbor_ids.T,
        W_user_age, W_user_level, W_item_id, W_item_cate, W_mem0, W_mem1)
    seq = jnp.concatenate([o_sa, o_sb], axis=-1).transpose(1, 0, 2)
    return (jnp.concatenate([o_ua, o_ub], axis=-1),
            jnp.concatenate([o_ia, o_ib], axis=-1),
            seq, o_u0, o_u1,
            o_n0.transpose(1, 0, 2), o_n1.transpose(1, 0, 2))


# speedup vs baseline: 1.7353x; 1.0497x over previous
import functools

import jax
import jax.numpy as jnp
from jax import lax
from jax.experimental import pallas as pl
from jax.experimental.pallas import tpu as pltpu
from jax.experimental.pallas import tpu_sc as plsc

B = 4096
L = 200
NN = 20
KV = 32
MEM = 32

NC = 2
NS = 16
NW = NC * NS
BC = B // NW


def _make_sc_kernel():
    mesh = plsc.VectorSubcoreMesh(core_axis_name="c", subcore_axis_name="s")

    out_type = [
        jax.ShapeDtypeStruct((B, KV), jnp.float32),
        jax.ShapeDtypeStruct((B, KV), jnp.float32),
        jax.ShapeDtypeStruct((B, KV), jnp.float32),
        jax.ShapeDtypeStruct((B, KV), jnp.float32),
        jax.ShapeDtypeStruct((L, B, KV), jnp.float32),
        jax.ShapeDtypeStruct((L, B, KV), jnp.float32),
        jax.ShapeDtypeStruct((B, MEM), jnp.float32),
        jax.ShapeDtypeStruct((B, MEM), jnp.float32),
        jax.ShapeDtypeStruct((NN, B, MEM), jnp.float32),
        jax.ShapeDtypeStruct((NN, B, MEM), jnp.float32),
    ]

    @functools.partial(
        pl.kernel,
        mesh=mesh,
        out_type=out_type,
        compiler_params=pltpu.CompilerParams(use_tc_tiling_on_sc=False),
        scratch_types=[
            pltpu.VMEM((8, BC), jnp.int32),
            pltpu.VMEM((NN, BC), jnp.int32),
            pltpu.VMEM((L, BC), jnp.int32),
            pltpu.VMEM((L, BC), jnp.int32),
        ] + [pltpu.VMEM((BC, KV), jnp.float32)] * 8 + [
            pltpu.SemaphoreType.DMA,
            pltpu.SemaphoreType.DMA,
        ],
    )
    def sc_kernel(ua, ul, ii, ic, sid, sct, uid, nb,
                  t_age, t_lvl, t_item, t_cate, t_m0, t_m1,
                  o_ua, o_ub, o_ia, o_ib, o_sa, o_sb, o_u0, o_u1, o_n0, o_n1,
                  ism, inb, isid, isct,
                  r0, r1, r2, r3, r4, r5, r6, r7, sg, sw):
        wid = lax.axis_index("s") * NC + lax.axis_index("c")
        b0 = wid * BC

        pltpu.sync_copy(ua.at[pl.ds(b0, BC)], ism.at[0])
        pltpu.sync_copy(ul.at[pl.ds(b0, BC)], ism.at[1])
        pltpu.sync_copy(ii.at[pl.ds(b0, BC)], ism.at[2])
        pltpu.sync_copy(ic.at[pl.ds(b0, BC)], ism.at[3])
        pltpu.sync_copy(uid.at[pl.ds(b0, BC)], ism.at[4])
        pltpu.sync_copy(nb.at[:, pl.ds(b0, BC)], inb)
        pltpu.sync_copy(sid.at[:, pl.ds(b0, BC)], isid)
        pltpu.sync_copy(sct.at[:, pl.ds(b0, BC)], isct)

        bufs = (r0, r1, r2, r3, r4, r5, r6, r7)

        def wave(jobs):
            gs = [pltpu.async_copy(tab.at[idx], bufs[j], sg)
                  for j, (tab, idx, _dst) in enumerate(jobs)]
            for g in gs:
                g.wait()
            ws = [pltpu.async_copy(bufs[j], dst, sw)
                  for j, (_tab, _idx, dst) in enumerate(jobs)]
            for w in ws:
                w.wait()

        wave([
            (t_age, ism.at[0], o_ua.at[pl.ds(b0, BC)]),
            (t_lvl, ism.at[1], o_ub.at[pl.ds(b0, BC)]),
            (t_item, ism.at[2], o_ia.at[pl.ds(b0, BC)]),
            (t_cate, ism.at[3], o_ib.at[pl.ds(b0, BC)]),
            (t_m0, ism.at[4], o_u0.at[pl.ds(b0, BC)]),
            (t_m1, ism.at[4], o_u1.at[pl.ds(b0, BC)]),
        ])

        def nbody(k, carry):
            wave([(t_m0 if j % 2 == 0 else t_m1,
                   inb.at[4 * k + j // 2],
                   (o_n0 if j % 2 == 0 else o_n1)
                   .at[4 * k + j // 2].at[pl.ds(b0, BC)])
                  for j in range(8)])
            return carry
        lax.fori_loop(0, NN // 4, nbody, 0)

        def sbody(k, carry):
            wave([(t_item if j % 2 == 0 else t_cate,
                   (isid if j % 2 == 0 else isct).at[4 * k + j // 2],
                   (o_sa if j % 2 == 0 else o_sb)
                   .at[4 * k + j // 2].at[pl.ds(b0, BC)])
                  for j in range(8)])
            return carry
        lax.fori_loop(0, L // 4, sbody, 0)

    return sc_kernel


_SC_KERNEL = _make_sc_kernel()


def kernel(user_age, user_level, item_id, item_cate, item_id_seq,
           item_cate_seq, userid, neighbor_ids, W_user_age, W_user_level,
           W_item_id, W_item_cate, W_mem0, W_mem1):
    o_ua, o_ub, o_ia, o_ib, o_sa, o_sb, o_u0, o_u1, o_n0, o_n1 = _SC_KERNEL(
        user_age, user_level, item_id, item_cate,
        item_id_seq.T, item_cate_seq.T, userid, neighbor_ids.T,
        W_user_age, W_user_level, W_item_id, W_item_cate, W_mem0, W_mem1)
    seq = jnp.concatenate([o_sa, o_sb], axis=-1).transpose(1, 0, 2)
    return (jnp.concatenate([o_ua, o_ub], axis=-1),
            jnp.concatenate([o_ia, o_ib], axis=-1),
            seq, o_u0, o_u1,
            o_n0.transpose(1, 0, 2), o_n1.transpose(1, 0, 2))


# speedup vs baseline: 1.7527x; 1.0100x over previous
import functools

import jax
import jax.numpy as jnp
from jax import lax
from jax.experimental import pallas as pl
from jax.experimental.pallas import tpu as pltpu
from jax.experimental.pallas import tpu_sc as plsc

B = 4096
L = 200
NN = 20
KV = 32
MEM = 32

NC = 2
NS = 16
NW = NC * NS
BC = B // NW


def _make_sc_kernel():
    mesh = plsc.VectorSubcoreMesh(core_axis_name="c", subcore_axis_name="s")

    out_type = [
        jax.ShapeDtypeStruct((B, KV), jnp.float32),
        jax.ShapeDtypeStruct((B, KV), jnp.float32),
        jax.ShapeDtypeStruct((B, KV), jnp.float32),
        jax.ShapeDtypeStruct((B, KV), jnp.float32),
        jax.ShapeDtypeStruct((L, B, KV), jnp.float32),
        jax.ShapeDtypeStruct((L, B, KV), jnp.float32),
        jax.ShapeDtypeStruct((B, MEM), jnp.float32),
        jax.ShapeDtypeStruct((B, MEM), jnp.float32),
        jax.ShapeDtypeStruct((NN, B, MEM), jnp.float32),
        jax.ShapeDtypeStruct((NN, B, MEM), jnp.float32),
    ]

    @functools.partial(
        pl.kernel,
        mesh=mesh,
        out_type=out_type,
        compiler_params=pltpu.CompilerParams(use_tc_tiling_on_sc=False),
        scratch_types=[
            pltpu.VMEM((8, BC), jnp.int32),
            pltpu.VMEM((NN, BC), jnp.int32),
            pltpu.VMEM((L, BC), jnp.int32),
            pltpu.VMEM((L, BC), jnp.int32),
        ] + [pltpu.VMEM((BC, KV), jnp.float32)] * 8
          + [pltpu.SemaphoreType.DMA] * 9,
    )
    def sc_kernel(ua, ul, ii, ic, sid, sct, uid, nb,
                  t_age, t_lvl, t_item, t_cate, t_m0, t_m1,
                  o_ua, o_ub, o_ia, o_ib, o_sa, o_sb, o_u0, o_u1, o_n0, o_n1,
                  ism, inb, isid, isct,
                  r0, r1, r2, r3, r4, r5, r6, r7,
                  g0, g1, g2, g3, g4, g5, g6, g7, sw):
        wid = lax.axis_index("s") * NC + lax.axis_index("c")
        b0 = wid * BC

        pltpu.sync_copy(ua.at[pl.ds(b0, BC)], ism.at[0])
        pltpu.sync_copy(ul.at[pl.ds(b0, BC)], ism.at[1])
        pltpu.sync_copy(ii.at[pl.ds(b0, BC)], ism.at[2])
        pltpu.sync_copy(ic.at[pl.ds(b0, BC)], ism.at[3])
        pltpu.sync_copy(uid.at[pl.ds(b0, BC)], ism.at[4])
        pltpu.sync_copy(nb.at[:, pl.ds(b0, BC)], inb)
        pltpu.sync_copy(sid.at[:, pl.ds(b0, BC)], isid)
        pltpu.sync_copy(sct.at[:, pl.ds(b0, BC)], isct)

        bufs = (r0, r1, r2, r3, r4, r5, r6, r7)
        gsems = (g0, g1, g2, g3, g4, g5, g6, g7)

        def wave(jobs):
            gs = [pltpu.async_copy(tab.at[idx], bufs[j], gsems[j])
                  for j, (tab, idx, _dst) in enumerate(jobs)]
            ws = []
            for j, (_tab, _idx, dst) in enumerate(jobs):
                gs[j].wait()
                ws.append(pltpu.async_copy(bufs[j], dst, sw))
            for w in ws:
                w.wait()

        wave([
            (t_age, ism.at[0], o_ua.at[pl.ds(b0, BC)]),
            (t_lvl, ism.at[1], o_ub.at[pl.ds(b0, BC)]),
            (t_item, ism.at[2], o_ia.at[pl.ds(b0, BC)]),
            (t_cate, ism.at[3], o_ib.at[pl.ds(b0, BC)]),
            (t_m0, ism.at[4], o_u0.at[pl.ds(b0, BC)]),
            (t_m1, ism.at[4], o_u1.at[pl.ds(b0, BC)]),
        ])

        def nbody(k, carry):
            wave([(t_m0 if j % 2 == 0 else t_m1,
                   inb.at[4 * k + j // 2],
                   (o_n0 if j % 2 == 0 else o_n1)
                   .at[4 * k + j // 2].at[pl.ds(b0, BC)])
                  for j in range(8)])
            return carry
        lax.fori_loop(0, NN // 4, nbody, 0)

        def sbody(k, carry):
            wave([(t_item if j % 2 == 0 else t_cate,
                   (isid if j % 2 == 0 else isct).at[4 * k + j // 2],
                   (o_sa if j % 2 == 0 else o_sb)
                   .at[4 * k + j // 2].at[pl.ds(b0, BC)])
                  for j in range(8)])
            return carry
        lax.fori_loop(0, L // 4, sbody, 0)

    return sc_kernel


_SC_KERNEL = _make_sc_kernel()


def kernel(user_age, user_level, item_id, item_cate, item_id_seq,
           item_cate_seq, userid, neighbor_ids, W_user_age, W_user_level,
           W_item_id, W_item_cate, W_mem0, W_mem1):
    o_ua, o_ub, o_ia, o_ib, o_sa, o_sb, o_u0, o_u1, o_n0, o_n1 = _SC_KERNEL(
        user_age, user_level, item_id, item_cate,
        item_id_seq.T, item_cate_seq.T, userid, neighbor_ids.T,
        W_user_age, W_user_level, W_item_id, W_item_cate, W_mem0, W_mem1)
    seq = jnp.concatenate([o_sa, o_sb], axis=-1).transpose(1, 0, 2)
    return (jnp.concatenate([o_ua, o_ub], axis=-1),
            jnp.concatenate([o_ia, o_ib], axis=-1),
            seq, o_u0, o_u1,
            o_n0.transpose(1, 0, 2), o_n1.transpose(1, 0, 2))
